# Initial kernel scaffold; baseline (speedup 1.0000x reference)
import jax
import jax.numpy as jnp
from jax.experimental import pallas as pl


def kernel(x, img_ids, mask, Wq, bq, Wk, bk, Wv, bv, avgs, std_devs):
    raise NotImplementedError("write your pallas kernel here")


# speedup vs baseline: 10.8183x; 10.8183x over previous
import jax
import jax.numpy as jnp
from jax.experimental import pallas as pl

DIM = 768
GRID_DIM = 24.0
HIGHEST = jax.lax.Precision.HIGHEST


def _attn_body(x_ref, wq_ref, bq_ref, wk_ref, bk_ref, wv_ref, bv_ref,
               idx_ref, out_ref):
    S = x_ref.shape[1]
    xb = x_ref[0]
    q = jnp.dot(xb, wq_ref[...], preferred_element_type=jnp.float32) + bq_ref[...]
    k = jnp.dot(xb, wk_ref[...], preferred_element_type=jnp.float32) + bk_ref[...]
    v = jnp.dot(xb, wv_ref[...], preferred_element_type=jnp.float32) + bv_ref[...]
    A = jax.lax.dot_general(q, k, (((1,), (1,)), ((), ())),
                            preferred_element_type=jnp.float32)
    idxb = idx_ref[0]
    cols = jax.lax.broadcasted_iota(jnp.int32, (S, S), 1)
    ats = []
    for j in range(4):
        mj = (cols == idxb[:, j:j + 1]).astype(jnp.float32)
        ats.append(jnp.sum(A * mj, axis=1, keepdims=True))
    at = jnp.concatenate(ats, axis=1)
    m = jnp.max(at, axis=1, keepdims=True)
    e = jnp.exp(at - m)
    w = e / jnp.sum(e, axis=1, keepdims=True)
    wm = jnp.zeros((S, S), jnp.float32)
    for j in range(4):
        mj = (cols == idxb[:, j:j + 1]).astype(jnp.float32)
        wm = wm + w[:, j:j + 1] * mj
    out = jnp.dot(wm, v, preferred_element_type=jnp.float32,
                  precision=HIGHEST)
    rows = jax.lax.broadcasted_iota(jnp.int32, (S, DIM), 0)
    out_ref[0] = jnp.where(rows == 0, 1.0, out)


def kernel(x, img_ids, mask, Wq, bq, Wk, bk, Wv, bv, avgs, std_devs):
    B, S, D = x.shape
    eps = jax.random.normal(jax.random.key(1234), (B, 2), dtype=jnp.float32)
    sel_avg = avgs[img_ids]
    sel_std = std_devs[img_ids]
    key_x = (eps[:, 0:1] - sel_avg[:, 0, :]) / sel_std[:, 0, :]
    key_y = (eps[:, 1:2] - sel_avg[:, 1, :]) / sel_std[:, 1, :]
    kx1, kx2 = jnp.ceil(key_x), jnp.floor(key_x)
    ky1, ky2 = jnp.ceil(key_y), jnp.floor(key_y)
    idx = jnp.stack([
        GRID_DIM * ky1 + kx1,
        GRID_DIM * ky1 + kx2,
        GRID_DIM * ky2 + kx1,
        GRID_DIM * ky2 + kx2,
    ], axis=-1).astype(jnp.int32) % S
    idx_full = jnp.concatenate(
        [jnp.zeros((B, 1, 4), jnp.int32), idx], axis=1)

    out = pl.pallas_call(
        _attn_body,
        grid=(B,),
        in_specs=[
            pl.BlockSpec((1, S, D), lambda b: (b, 0, 0)),
            pl.BlockSpec((D, D), lambda b: (0, 0)),
            pl.BlockSpec((1, D), lambda b: (0, 0)),
            pl.BlockSpec((D, D), lambda b: (0, 0)),
            pl.BlockSpec((1, D), lambda b: (0, 0)),
            pl.BlockSpec((D, D), lambda b: (0, 0)),
            pl.BlockSpec((1, D), lambda b: (0, 0)),
            pl.BlockSpec((1, S, 4), lambda b: (b, 0, 0)),
        ],
        out_specs=pl.BlockSpec((1, S, D), lambda b: (b, 0, 0)),
        out_shape=jax.ShapeDtypeStruct((B, S, D), jnp.float32),
    )(x, Wq, bq.reshape(1, D), Wk, bk.reshape(1, D), Wv, bv.reshape(1, D),
      idx_full)
    return out


# speedup vs baseline: 18.8770x; 1.7449x over previous
import jax
import jax.numpy as jnp
from jax.experimental import pallas as pl
from jax.experimental.pallas import tpu as pltpu

DIM = 768
GRID_DIM = 24.0


def _attn_body(ids_ref, x_ref, wq_ref, bq_ref, wk_ref, bk_ref, wv_ref,
               bv_ref, avg_ref, std_ref, eps_ref, out_ref):
    S = x_ref.shape[1]
    xb = x_ref[0]
    q = jnp.dot(xb, wq_ref[...], preferred_element_type=jnp.float32) + bq_ref[...]
    k = jnp.dot(xb, wk_ref[...], preferred_element_type=jnp.float32) + bk_ref[...]
    v = jnp.dot(xb, wv_ref[...], preferred_element_type=jnp.float32) + bv_ref[...]
    AT = jax.lax.dot_general(k, q, (((1,), (1,)), ((), ())),
                             preferred_element_type=jnp.float32)

    key_x = (eps_ref[0, :, 0:1] - avg_ref[0, 0:1, :]) / std_ref[0, 0:1, :]
    key_y = (eps_ref[0, :, 1:2] - avg_ref[0, 1:2, :]) / std_ref[0, 1:2, :]
    kx1, kx2 = jnp.ceil(key_x), jnp.floor(key_x)
    ky1, ky2 = jnp.ceil(key_y), jnp.floor(key_y)
    zero = jnp.zeros((1, 1), jnp.int32)
    idxs = []
    for fy, fx in ((ky1, kx1), (ky1, kx2), (ky2, kx1), (ky2, kx2)):
        ij = (GRID_DIM * fy + fx).astype(jnp.int32) % S
        idxs.append(jnp.concatenate([zero, ij], axis=1))

    rows = jax.lax.broadcasted_iota(jnp.int32, (S, S), 0)
    ats = []
    for j in range(4):
        mj = (rows == idxs[j]).astype(jnp.float32)
        ats.append(jnp.sum(AT * mj, axis=0, keepdims=True))
    at = jnp.concatenate(ats, axis=0)
    m = jnp.max(at, axis=0, keepdims=True)
    e = jnp.exp(at - m)
    w = e / jnp.sum(e, axis=0, keepdims=True)
    wt = jnp.zeros((S, S), jnp.float32)
    for j in range(4):
        mj = (rows == idxs[j]).astype(jnp.float32)
        wt = wt + w[j:j + 1, :] * mj
    out = jax.lax.dot_general(wt, v, (((0,), (0,)), ((), ())),
                              preferred_element_type=jnp.float32)
    out_rows = jax.lax.broadcasted_iota(jnp.int32, (S, DIM), 0)
    out_ref[0] = jnp.where(out_rows == 0, 1.0, out)


def kernel(x, img_ids, mask, Wq, bq, Wk, bk, Wv, bv, avgs, std_devs):
    B, S, D = x.shape
    P = S - 1
    eps = jax.random.normal(jax.random.key(1234), (B, 2), dtype=jnp.float32)

    grid_spec = pltpu.PrefetchScalarGridSpec(
        num_scalar_prefetch=1,
        grid=(B,),
        in_specs=[
            pl.BlockSpec((1, S, D), lambda b, ids: (b, 0, 0)),
            pl.BlockSpec((D, D), lambda b, ids: (0, 0)),
            pl.BlockSpec((1, D), lambda b, ids: (0, 0)),
            pl.BlockSpec((D, D), lambda b, ids: (0, 0)),
            pl.BlockSpec((1, D), lambda b, ids: (0, 0)),
            pl.BlockSpec((D, D), lambda b, ids: (0, 0)),
            pl.BlockSpec((1, D), lambda b, ids: (0, 0)),
            pl.BlockSpec((1, 2, P), lambda b, ids: (ids[b], 0, 0)),
            pl.BlockSpec((1, 2, P), lambda b, ids: (ids[b], 0, 0)),
            pl.BlockSpec((1, 1, 2), lambda b, ids: (b, 0, 0)),
        ],
        out_specs=pl.BlockSpec((1, S, D), lambda b, ids: (b, 0, 0)),
    )
    out = pl.pallas_call(
        _attn_body,
        grid_spec=grid_spec,
        out_shape=jax.ShapeDtypeStruct((B, S, D), jnp.float32),
    )(img_ids, x, Wq, bq.reshape(1, D), Wk, bk.reshape(1, D), Wv,
      bv.reshape(1, D), avgs, std_devs, eps.reshape(B, 1, 2))
    return out
